# Initial kernel scaffold; baseline (speedup 1.0000x reference)
import jax
import jax.numpy as jnp
from jax.experimental import pallas as pl


def kernel(x, edge_index, edge_attr, batch, params):
    raise NotImplementedError("write your pallas kernel here")


# speedup vs baseline: 1.2885x; 1.2885x over previous
import functools

import jax
import jax.numpy as jnp
from jax.experimental import pallas as pl
from jax.experimental.pallas import tpu as pltpu

N = 10000
IN = 128
MID = 240
MIDP = 256
FC = 40
OUT = 2
G = 64


def _dot3(x, w, dn=None):
    xh = x.astype(jnp.bfloat16)
    xl = (x - xh.astype(jnp.float32)).astype(jnp.bfloat16)
    wh = w.astype(jnp.bfloat16)
    wl = (w - wh.astype(jnp.float32)).astype(jnp.bfloat16)
    if dn is None:
        d = lambda a, b: jnp.dot(a, b, preferred_element_type=jnp.float32)
    else:
        d = lambda a, b: jax.lax.dot_general(a, b, dn,
                                             preferred_element_type=jnp.float32)
    return d(xh, wh) + d(xh, wl) + d(xl, wh)


def _bn(h, g, b):
    mu = jnp.mean(h, axis=0, keepdims=True)
    var = jnp.mean((h - mu) ** 2, axis=0, keepdims=True)
    return g * (h - mu) * jax.lax.rsqrt(var + 1e-5) + b


def _lrelu(x):
    return jnp.where(x >= 0, x, 0.01 * x)


def _tc_in_body(x_ref, g_ref, be_ref, w_ref, u_ref):
    h = _bn(x_ref[...], g_ref[...], be_ref[...])
    u = jnp.dot(h, w_ref[...], preferred_element_type=jnp.float32)
    u_ref[...] = jnp.pad(u, ((0, 0), (0, MIDP - MID)))


def _tc_mid_body(m_ref, b_ref, g_ref, be_ref, w_ref, un_ref):
    m = m_ref[...][:, :MID]
    a = _lrelu(m + b_ref[...])
    h = _bn(a, g_ref[...], be_ref[...])
    un = jnp.dot(h, w_ref[...], preferred_element_type=jnp.float32)
    un_ref[...] = jnp.pad(un, ((0, 0), (0, MIDP - MID)))


def _tc_fin_body(m_ref, b_ref, g_ref, be_ref, batch_ref,
                 wa_ref, ba_ref, g0_ref, be0_ref,
                 wf1_ref, bf1_ref, g4_ref, be4_ref,
                 wf2_ref, bf2_ref, g5_ref, be5_ref,
                 wf3_ref, bf3_ref, z_ref):
    m = m_ref[...][:, :MID]
    a = _lrelu(m + b_ref[...])
    h = _bn(a, g_ref[...], be_ref[...])
    s = jnp.dot(h, wa_ref[...], preferred_element_type=jnp.float32) + ba_ref[...]
    oh = (batch_ref[...] == jax.lax.broadcasted_iota(jnp.int32, (1, G), 1)
          ).astype(jnp.float32)
    mg = jnp.max(jnp.where(oh > 0, s, -1e30), axis=0, keepdims=True)
    row_max = _dot3(oh, mg.T)
    e = jnp.exp(s - row_max)
    zsum = _dot3(oh, e, (((0,), (0,)), ((), ())))
    row_z = _dot3(oh, zsum)
    att = e / (row_z + 1e-16)
    pooled = _dot3(oh, h * att, (((0,), (0,)), ((), ())))
    q = _bn(pooled, g0_ref[...], be0_ref[...])
    q = _lrelu(jnp.dot(q, wf1_ref[...], preferred_element_type=jnp.float32)
               + bf1_ref[...])
    q = _bn(q, g4_ref[...], be4_ref[...])
    q = _lrelu(jnp.dot(q, wf2_ref[...], preferred_element_type=jnp.float32)
               + bf2_ref[...])
    q = _bn(q, g5_ref[...], be5_ref[...])
    z_ref[...] = (jnp.dot(q, wf3_ref[...], preferred_element_type=jnp.float32)
                  + bf3_ref[...])


def _call_tc(body, out_shape, *args):
    return pl.pallas_call(
        body,
        out_shape=jax.ShapeDtypeStruct(*out_shape),
    )(*args)


def kernel(x, edge_index, edge_attr, batch, params):
    p = params
    row, col = edge_index[0], edge_index[1]

    r2 = lambda v: v.reshape(1, -1)

    deg = 1.0 + jax.ops.segment_sum(edge_attr, col, num_segments=N)
    dis = deg ** -0.5
    dis2 = (dis * dis).reshape(N, 1)
    norm = dis[row] * edge_attr * dis[col]

    def spmm(u256):
        u = u256[:, :MID]
        m = jax.ops.segment_sum(u[row] * norm[:, None], col, num_segments=N)
        m = m + dis2 * u
        return jnp.pad(m, ((0, 0), (0, MIDP - MID)))

    u = _call_tc(_tc_in_body, ((N, MIDP), jnp.float32),
                 x, r2(p['g_in']), r2(p['be_in']), p['W1'])

    for Wn, b, g, be in ((p['W2'], p['b1'], p['g1'], p['be1']),
                         (p['W3'], p['b2'], p['g2'], p['be2'])):
        m = spmm(u)
        u = _call_tc(_tc_mid_body, ((N, MIDP), jnp.float32),
                     m, r2(b), r2(g), r2(be), Wn)

    m = spmm(u)
    z = _call_tc(_tc_fin_body, ((G, OUT), jnp.float32),
                 m, r2(p['b3']), r2(p['g3']), r2(p['be3']),
                 batch.reshape(N, 1),
                 p['Wa'], r2(p['ba']), r2(p['g0']), r2(p['be0']),
                 p['Wf1'], r2(p['bf1']), r2(p['g4']), r2(p['be4']),
                 p['Wf2'], r2(p['bf2']), r2(p['g5']), r2(p['be5']),
                 p['Wf3'], r2(p['bf3']))
    return z


# speedup vs baseline: 3.9401x; 3.0580x over previous
import dataclasses
import functools

import jax
import jax.numpy as jnp
from jax.experimental import pallas as pl
from jax.experimental.pallas import tpu as pltpu
from jax.experimental.pallas import tpu_sc as plsc

N = 10000
IN = 128
MID = 240
MIDP = 256
FC = 40
OUT = 2
G = 64

NP_ = 10240
CH = 128
NCH = 158
EP = 16 * NCH * CH
EW = EP // 32
ND = NP_ // 32

_mesh = plsc.VectorSubcoreMesh(core_axis_name="c", subcore_axis_name="s")

_SC_CP = pltpu.CompilerParams()
if "needs_layout_passes" in pltpu.CompilerParams.__dataclass_fields__:
    _SC_CP = dataclasses.replace(_SC_CP, needs_layout_passes=False)


def _dot3(x, w, dn=None):
    xh = x.astype(jnp.bfloat16)
    xl = (x - xh.astype(jnp.float32)).astype(jnp.bfloat16)
    wh = w.astype(jnp.bfloat16)
    wl = (w - wh.astype(jnp.float32)).astype(jnp.bfloat16)
    if dn is None:
        d = lambda a, b: jnp.dot(a, b, preferred_element_type=jnp.float32)
    else:
        d = lambda a, b: jax.lax.dot_general(a, b, dn,
                                             preferred_element_type=jnp.float32)
    return d(xh, wh) + d(xh, wl) + d(xl, wh)


def _bn(h, g, b):
    mu = jnp.mean(h, axis=0, keepdims=True)
    var = jnp.mean((h - mu) ** 2, axis=0, keepdims=True)
    return g * (h - mu) * jax.lax.rsqrt(var + 1e-5) + b


def _lrelu(x):
    return jnp.where(x >= 0, x, 0.01 * x)


def _qrsqrt(x):
    xi = jax.lax.bitcast_convert_type(x, jnp.int32)
    yi = jnp.int32(0x5F3759DF) - (xi >> 1)
    y = jax.lax.bitcast_convert_type(yi, jnp.float32)
    y = y * (1.5 - 0.5 * x * y * y)
    y = y * (1.5 - 0.5 * x * y * y)
    y = y * (1.5 - 0.5 * x * y * y)
    return y


_GDN = jax.lax.GatherDimensionNumbers(
    offset_dims=(), collapsed_slice_dims=(0,), start_index_map=(0,))


def _bcast16(v16, i):
    idx = (jnp.zeros((16,), jnp.int32) + i).reshape(16, 1)
    return jax.lax.gather(v16, idx, _GDN, (1,),
                          mode=jax.lax.GatherScatterMode.PROMISE_IN_BOUNDS)


def _bcast16i(v16, i):
    idx = (jnp.zeros((16,), jnp.int32) + i).reshape(16, 1)
    return jax.lax.gather(v16, idx, _GDN, (1,),
                          mode=jax.lax.GatherScatterMode.PROMISE_IN_BOUNDS)


def _z16f():
    return jnp.zeros((16,), jnp.float32)


NB = NP_ // 16
ECH = 512
CAPW = 16384
EH = EP // 2
NSCH = EH // ECH


@functools.partial(
    pl.kernel,
    out_type=[jax.ShapeDtypeStruct((2, 16, CAPW), jnp.int32),
              jax.ShapeDtypeStruct((2, 16, CAPW), jnp.int32),
              jax.ShapeDtypeStruct((2, 16, CAPW), jnp.float32),
              jax.ShapeDtypeStruct((2, 16, 16), jnp.int32),
              jax.ShapeDtypeStruct((NP_,), jnp.float32)],
    mesh=_mesh,
    compiler_params=_SC_CP,
    scratch_types=[
        pltpu.VMEM((ECH,), jnp.int32),
        pltpu.VMEM((ECH,), jnp.int32),
        pltpu.VMEM((ECH,), jnp.float32),
        pltpu.VMEM((16, NB), jnp.float32),
        pltpu.VMEM((NB,), jnp.float32),
        pltpu.VMEM((NP_,), jnp.float32),
        pltpu.VMEM((CAPW,), jnp.int32),
        pltpu.VMEM((CAPW,), jnp.int32),
        pltpu.VMEM((CAPW,), jnp.float32),
        pltpu.VMEM((16,), jnp.int32),
        pltpu.VMEM((ND,), jnp.float32),
        pltpu.VMEM_SHARED((NP_,), jnp.float32),
    ],
)
def _sc_pre(r_hbm, c_hbm, w_hbm,
            rb_hbm, cb_hbm, nb_hbm, cnt_hbm, dis2_hbm,
            rv_v, cv_v, wv_v, acc_v, dt_v, dis_v,
            sr_v, sc_v, sn_v, cnt_v, d2_v, dis_sh):
    cid = jax.lax.axis_index("c")
    sid = jax.lax.axis_index("s")
    wid = cid * 16 + sid
    lo = sid * NB
    lanes = jnp.arange(16, dtype=jnp.int32)

    @pl.loop(0, 16)
    def _zl(l):
        @pl.loop(0, NB // 16)
        def _zg(g):
            acc_v[l, pl.ds(g * 16, 16)] = _z16f()

    z16i = jnp.zeros((16,), jnp.int32)

    @pl.loop(0, CAPW // 16)
    def _zs(g):
        sl = pl.ds(g * 16, 16)
        sr_v[sl] = z16i
        sc_v[sl] = z16i
        sn_v[sl] = _z16f()

    @pl.loop(0, EP // ECH)
    def _deg(k):
        e0 = k * ECH
        pltpu.sync_copy(c_hbm.at[pl.ds(e0, ECH)], cv_v)
        pltpu.sync_copy(w_hbm.at[pl.ds(e0, ECH)], wv_v)

        @pl.loop(0, ECH // 16, step=4)
        def _g(g0):
            for gg in range(4):
                g = g0 + gg
                c16 = cv_v[pl.ds(g * 16, 16)]
                w16 = wv_v[pl.ds(g * 16, 16)]
                rel = c16 - lo
                msk = (rel >= 0) & (rel < NB)
                idxc = jnp.clip(rel, 0, NB - 1)
                plsc.addupdate_scatter(acc_v, [lanes, idxc], w16, mask=msk)

    @pl.loop(0, NB // 16)
    def _dis(g):
        sl = pl.ds(g * 16, 16)
        tot = acc_v[0, sl]
        for l in range(1, 16):
            tot = tot + acc_v[l, sl]
        dt_v[sl] = _qrsqrt(tot + 1.0)

    pltpu.sync_copy(dt_v, dis_sh.at[pl.ds(lo, NB)])
    plsc.subcore_barrier()
    pltpu.sync_copy(dis_sh, dis_v)

    def _chunk(k, fill):
        e0 = cid * EH + k * ECH
        pltpu.sync_copy(r_hbm.at[pl.ds(e0, ECH)], rv_v)
        pltpu.sync_copy(c_hbm.at[pl.ds(e0, ECH)], cv_v)
        pltpu.sync_copy(w_hbm.at[pl.ds(e0, ECH)], wv_v)

        def _grp(g, fill):
            sl = pl.ds(g * 16, 16)
            r16 = rv_v[sl]
            c16 = cv_v[sl]
            w16 = wv_v[sl]
            rel = c16 - lo
            msk = (rel >= 0) & (rel < NB)
            mi = msk.astype(jnp.int32)
            csum = plsc.cumsum(mi)
            pos = jnp.clip(fill + csum - mi, 0, CAPW - 1)
            nrm = plsc.load_gather(dis_v, [r16]) * w16 \
                * plsc.load_gather(dis_v, [c16])
            plsc.store_scatter(sr_v, [pos], r16, mask=msk)
            plsc.store_scatter(sc_v, [pos], jnp.clip(rel, 0, NB - 1),
                               mask=msk)
            plsc.store_scatter(sn_v, [pos], nrm, mask=msk)
            return fill + _bcast16i(csum, 15)

        return jax.lax.fori_loop(0, ECH // 16, _grp, fill)

    fill = jax.lax.fori_loop(0, NSCH, _chunk,
                             jnp.zeros((16,), jnp.int32))

    cnt_v[pl.ds(0, 16)] = jnp.where(lanes == 0, fill, 0)
    pltpu.sync_copy(cnt_v, cnt_hbm.at[cid].at[sid])
    pltpu.sync_copy(sr_v, rb_hbm.at[cid].at[sid])
    pltpu.sync_copy(sc_v, cb_hbm.at[cid].at[sid])
    pltpu.sync_copy(sn_v, nb_hbm.at[cid].at[sid])

    n0 = wid * ND

    @pl.loop(0, ND // 16)
    def _d2(g):
        d16 = dis_v[pl.ds(n0 + g * 16, 16)]
        d2_v[pl.ds(g * 16, 16)] = d16 * d16

    pltpu.sync_copy(d2_v, dis2_hbm.at[pl.ds(n0, ND)])


@functools.partial(
    pl.kernel,
    out_type=jax.ShapeDtypeStruct((2, NP_, 128), jnp.float32),
    mesh=_mesh,
    compiler_params=_SC_CP,
    scratch_types=[
        pltpu.VMEM((CH,), jnp.int32),
        pltpu.VMEM((CH,), jnp.int32),
        pltpu.VMEM((CH,), jnp.float32),
        pltpu.VMEM((CH, 128), jnp.float32),
        pltpu.VMEM((128, 128), jnp.float32),
        pltpu.VMEM((NB,), jnp.float32),
        pltpu.VMEM((16,), jnp.int32),
        pltpu.VMEM_SHARED((NP_, 128), jnp.float32),
        pltpu.SemaphoreType.DMA,
    ],
)
def _sc_spmm(u_hbm, rb_hbm, cb_hbm, nb_hbm, cnt_hbm, dis2_hbm,
             m_hbm,
             ridx_v, cidx_v, nrm_v, rows_v, ioa_v, d2_v, cnt_v, acc_sh,
             sem):
    cid = jax.lax.axis_index("c")
    sid = jax.lax.axis_index("s")
    row0 = sid * NB


    @pl.loop(0, 128)
    def _z(i):
        @pl.loop(0, 8)
        def _zj(j):
            ioa_v[i, pl.ds(j * 16, 16)] = _z16f()

    @pl.loop(0, 5)
    def _zc(blk):
        pltpu.sync_copy(ioa_v, acc_sh.at[pl.ds(row0 + blk * 128, 128)])

    for half in range(2):
        pltpu.sync_copy(cnt_hbm.at[half].at[sid], cnt_v)
        nch = (jnp.sum(cnt_v[pl.ds(0, 16)]) + (CH - 1)) // CH

        @pl.loop(0, nch)
        def _edge(k):
            e0 = k * CH
            pltpu.sync_copy(rb_hbm.at[half].at[sid].at[pl.ds(e0, CH)],
                            ridx_v)
            pltpu.sync_copy(cb_hbm.at[half].at[sid].at[pl.ds(e0, CH)],
                            cidx_v)
            pltpu.sync_copy(nb_hbm.at[half].at[sid].at[pl.ds(e0, CH)],
                            nrm_v)
            pltpu.async_copy(u_hbm.at[cid].at[ridx_v], rows_v, sem).wait()

            @pl.loop(0, 8)
            def _ofs(g):
                sl = pl.ds(g * 16, 16)
                cidx_v[sl] = cidx_v[sl] + row0

            @pl.loop(0, 8)
            def _g(g):
                n16 = nrm_v[pl.ds(g * 16, 16)]

                @pl.loop(0, 16)
                def _b(b):
                    sc = _bcast16(n16, b)
                    row = g * 16 + b
                    for j in range(8):
                        sl = pl.ds(j * 16, 16)
                        rows_v[row, sl] = rows_v[row, sl] * sc

            pltpu.sync_copy(rows_v, acc_sh.at[cidx_v], add=True)

    pltpu.sync_copy(dis2_hbm.at[pl.ds(row0, NB)], d2_v)

    @pl.loop(0, 5)
    def _out(blk):
        r0 = blk * 128
        pltpu.sync_copy(acc_sh.at[pl.ds(row0 + r0, 128)], ioa_v)
        pltpu.sync_copy(u_hbm.at[cid].at[pl.ds(row0 + r0, 128)], rows_v)

        @pl.loop(0, 128)
        def _r(i):
            d2b = plsc.load_gather(
                d2_v, [jnp.zeros((16,), jnp.int32) + (r0 + i)])
            for j in range(8):
                sl = pl.ds(j * 16, 16)
                ioa_v[i, sl] = ioa_v[i, sl] + d2b * rows_v[i, sl]

        pltpu.sync_copy(ioa_v, m_hbm.at[cid].at[pl.ds(row0 + r0, 128)])


def _tc_in_body(x_ref, g_ref, be_ref, w_ref, u_ref):
    h = _bn(x_ref[...], g_ref[...], be_ref[...])
    u = jnp.dot(h, w_ref[...], preferred_element_type=jnp.float32)
    u_ref[0] = jnp.pad(u[:, :128], ((0, NP_ - N), (0, 0)))
    u_ref[1] = jnp.pad(u[:, 128:], ((0, NP_ - N), (0, MIDP - MID)))


def _tc_mid_body(m_ref, b_ref, g_ref, be_ref, w_ref, un_ref):
    m = jnp.concatenate([m_ref[0][:N], m_ref[1][:N, :MID - 128]], axis=1)
    a = _lrelu(m + b_ref[...])
    h = _bn(a, g_ref[...], be_ref[...])
    un = jnp.dot(h, w_ref[...], preferred_element_type=jnp.float32)
    un_ref[0] = jnp.pad(un[:, :128], ((0, NP_ - N), (0, 0)))
    un_ref[1] = jnp.pad(un[:, 128:], ((0, NP_ - N), (0, MIDP - MID)))


def _tc_fin_body(m_ref, b_ref, g_ref, be_ref, batch_ref,
                 wa_ref, ba_ref, g0_ref, be0_ref,
                 wf1_ref, bf1_ref, g4_ref, be4_ref,
                 wf2_ref, bf2_ref, g5_ref, be5_ref,
                 wf3_ref, bf3_ref, z_ref):
    m = jnp.concatenate([m_ref[0][:N], m_ref[1][:N, :MID - 128]], axis=1)
    a = _lrelu(m + b_ref[...])
    h = _bn(a, g_ref[...], be_ref[...])
    s = jnp.dot(h, wa_ref[...], preferred_element_type=jnp.float32) + ba_ref[...]
    oh = (batch_ref[...] == jax.lax.broadcasted_iota(jnp.int32, (1, G), 1)
          ).astype(jnp.float32)
    mg = jnp.max(jnp.where(oh > 0, s, -1e30), axis=0, keepdims=True)
    row_max = _dot3(oh, mg.T)
    e = jnp.exp(s - row_max)
    zsum = _dot3(oh, e, (((0,), (0,)), ((), ())))
    row_z = _dot3(oh, zsum)
    att = e / (row_z + 1e-16)
    pooled = _dot3(oh, h * att, (((0,), (0,)), ((), ())))
    q = _bn(pooled, g0_ref[...], be0_ref[...])
    q = _lrelu(jnp.dot(q, wf1_ref[...], preferred_element_type=jnp.float32)
               + bf1_ref[...])
    q = _bn(q, g4_ref[...], be4_ref[...])
    q = _lrelu(jnp.dot(q, wf2_ref[...], preferred_element_type=jnp.float32)
               + bf2_ref[...])
    q = _bn(q, g5_ref[...], be5_ref[...])
    z_ref[...] = (jnp.dot(q, wf3_ref[...], preferred_element_type=jnp.float32)
                  + bf3_ref[...])


def _call_tc(body, out_shape, *args):
    return pl.pallas_call(
        body,
        out_shape=jax.ShapeDtypeStruct(*out_shape),
    )(*args)


def kernel(x, edge_index, edge_attr, batch, params):
    p = params
    row, col = edge_index[0], edge_index[1]

    r2 = lambda v: v.reshape(1, -1)

    npad = EP - row.shape[0]
    fill = (jnp.arange(npad, dtype=jnp.int32) * 7) % N
    r_pad = jnp.concatenate([row, fill])
    c_pad = jnp.concatenate([col, (fill * 13) % N])
    w_pad = jnp.concatenate([edge_attr, jnp.zeros((npad,), jnp.float32)])

    rb, cb, nb, cnt, dis2 = _sc_pre(r_pad, c_pad, w_pad)

    u = _call_tc(_tc_in_body, ((2, NP_, 128), jnp.float32),
                 x, r2(p['g_in']), r2(p['be_in']), p['W1'])

    for Wn, b, g, be in ((p['W2'], p['b1'], p['g1'], p['be1']),
                         (p['W3'], p['b2'], p['g2'], p['be2'])):
        m = _sc_spmm(u, rb, cb, nb, cnt, dis2)
        u = _call_tc(_tc_mid_body, ((2, NP_, 128), jnp.float32),
                     m, r2(b), r2(g), r2(be), Wn)

    m = _sc_spmm(u, rb, cb, nb, cnt, dis2)
    z = _call_tc(_tc_fin_body, ((G, OUT), jnp.float32),
                 m, r2(p['b3']), r2(p['g3']), r2(p['be3']),
                 batch.reshape(N, 1),
                 p['Wa'], r2(p['ba']), r2(p['g0']), r2(p['be0']),
                 p['Wf1'], r2(p['bf1']), r2(p['g4']), r2(p['be4']),
                 p['Wf2'], r2(p['bf2']), r2(p['g5']), r2(p['be5']),
                 p['Wf3'], r2(p['bf3']))
    return z


# speedup vs baseline: 6.1843x; 1.5696x over previous
import dataclasses
import functools

import jax
import jax.numpy as jnp
from jax.experimental import pallas as pl
from jax.experimental.pallas import tpu as pltpu
from jax.experimental.pallas import tpu_sc as plsc

N = 10000
IN = 128
MID = 240
MIDP = 256
FC = 40
OUT = 2
G = 64

NP_ = 10240
CH = 128
NCH = 158
EP = 16 * NCH * CH
EW = EP // 32
ND = NP_ // 32

_mesh = plsc.VectorSubcoreMesh(core_axis_name="c", subcore_axis_name="s")

_SC_CP = pltpu.CompilerParams()
if "needs_layout_passes" in pltpu.CompilerParams.__dataclass_fields__:
    _SC_CP = dataclasses.replace(_SC_CP, needs_layout_passes=False)


def _dot3(x, w, dn=None):
    xh = x.astype(jnp.bfloat16)
    xl = (x - xh.astype(jnp.float32)).astype(jnp.bfloat16)
    wh = w.astype(jnp.bfloat16)
    wl = (w - wh.astype(jnp.float32)).astype(jnp.bfloat16)
    if dn is None:
        d = lambda a, b: jnp.dot(a, b, preferred_element_type=jnp.float32)
    else:
        d = lambda a, b: jax.lax.dot_general(a, b, dn,
                                             preferred_element_type=jnp.float32)
    return d(xh, wh) + d(xh, wl) + d(xl, wh)


def _bn(h, g, b):
    mu = jnp.mean(h, axis=0, keepdims=True)
    var = jnp.mean((h - mu) ** 2, axis=0, keepdims=True)
    return g * (h - mu) * jax.lax.rsqrt(var + 1e-5) + b


def _lrelu(x):
    return jnp.where(x >= 0, x, 0.01 * x)


def _qrsqrt(x):
    xi = jax.lax.bitcast_convert_type(x, jnp.int32)
    yi = jnp.int32(0x5F3759DF) - (xi >> 1)
    y = jax.lax.bitcast_convert_type(yi, jnp.float32)
    y = y * (1.5 - 0.5 * x * y * y)
    y = y * (1.5 - 0.5 * x * y * y)
    y = y * (1.5 - 0.5 * x * y * y)
    return y


_GDN = jax.lax.GatherDimensionNumbers(
    offset_dims=(), collapsed_slice_dims=(0,), start_index_map=(0,))


def _bcast16(v16, i):
    idx = (jnp.zeros((16,), jnp.int32) + i).reshape(16, 1)
    return jax.lax.gather(v16, idx, _GDN, (1,),
                          mode=jax.lax.GatherScatterMode.PROMISE_IN_BOUNDS)


def _bcast16i(v16, i):
    idx = (jnp.zeros((16,), jnp.int32) + i).reshape(16, 1)
    return jax.lax.gather(v16, idx, _GDN, (1,),
                          mode=jax.lax.GatherScatterMode.PROMISE_IN_BOUNDS)


def _z16f():
    return jnp.zeros((16,), jnp.float32)


NB = NP_ // 16
ECH = 2048
CAPW = 16384
EH = EP // 2
NSCH = EH // ECH


@functools.partial(
    pl.kernel,
    out_type=[jax.ShapeDtypeStruct((2, 16, CAPW), jnp.int32),
              jax.ShapeDtypeStruct((2, 16, CAPW), jnp.int32),
              jax.ShapeDtypeStruct((2, 16, CAPW), jnp.float32),
              jax.ShapeDtypeStruct((2, 16, 16), jnp.int32),
              jax.ShapeDtypeStruct((NP_,), jnp.float32)],
    mesh=_mesh,
    compiler_params=_SC_CP,
    scratch_types=[
        pltpu.VMEM((ECH,), jnp.int32),
        pltpu.VMEM((ECH,), jnp.int32),
        pltpu.VMEM((ECH,), jnp.float32),
        pltpu.VMEM((16, NB), jnp.float32),
        pltpu.VMEM((NB,), jnp.float32),
        pltpu.VMEM((NP_,), jnp.float32),
        pltpu.VMEM((CAPW,), jnp.int32),
        pltpu.VMEM((CAPW,), jnp.int32),
        pltpu.VMEM((CAPW,), jnp.float32),
        pltpu.VMEM((16,), jnp.int32),
        pltpu.VMEM((ND,), jnp.float32),
        pltpu.VMEM_SHARED((NP_,), jnp.float32),
        pltpu.SemaphoreType.DMA,
    ],
)
def _sc_pre(r_hbm, c_hbm, w_hbm,
            rb_hbm, cb_hbm, nb_hbm, cnt_hbm, dis2_hbm,
            rv_v, cv_v, wv_v, acc_v, dt_v, dis_v,
            sr_v, sc_v, sn_v, cnt_v, d2_v, dis_sh, dsem):
    cid = jax.lax.axis_index("c")
    sid = jax.lax.axis_index("s")
    wid = cid * 16 + sid
    lo = sid * NB
    lanes = jnp.arange(16, dtype=jnp.int32)

    @pl.loop(0, 16)
    def _zl(l):
        @pl.loop(0, NB // 16)
        def _zg(g):
            acc_v[l, pl.ds(g * 16, 16)] = _z16f()

    z16i = jnp.zeros((16,), jnp.int32)

    @pl.loop(0, CAPW // 16)
    def _zs(g):
        sl = pl.ds(g * 16, 16)
        sr_v[sl] = z16i
        sc_v[sl] = z16i
        sn_v[sl] = _z16f()

    @pl.loop(0, EP // ECH)
    def _deg(k):
        e0 = k * ECH
        d1 = pltpu.async_copy(c_hbm.at[pl.ds(e0, ECH)], cv_v, dsem)
        d2 = pltpu.async_copy(w_hbm.at[pl.ds(e0, ECH)], wv_v, dsem)
        d1.wait()
        d2.wait()

        @pl.loop(0, ECH // 16, step=4)
        def _g(g0):
            for gg in range(4):
                g = g0 + gg
                c16 = cv_v[pl.ds(g * 16, 16)]
                w16 = wv_v[pl.ds(g * 16, 16)]
                rel = c16 - lo
                msk = (rel >= 0) & (rel < NB)
                idxc = jnp.clip(rel, 0, NB - 1)
                plsc.addupdate_scatter(acc_v, [lanes, idxc], w16, mask=msk)

    @pl.loop(0, NB // 16)
    def _dis(g):
        sl = pl.ds(g * 16, 16)
        tot = acc_v[0, sl]
        for l in range(1, 16):
            tot = tot + acc_v[l, sl]
        dt_v[sl] = _qrsqrt(tot + 1.0)

    pltpu.sync_copy(dt_v, dis_sh.at[pl.ds(lo, NB)])
    plsc.subcore_barrier()
    pltpu.sync_copy(dis_sh, dis_v)

    def _chunk(k, fill):
        e0 = cid * EH + k * ECH
        d1 = pltpu.async_copy(r_hbm.at[pl.ds(e0, ECH)], rv_v, dsem)
        d2 = pltpu.async_copy(c_hbm.at[pl.ds(e0, ECH)], cv_v, dsem)
        d3 = pltpu.async_copy(w_hbm.at[pl.ds(e0, ECH)], wv_v, dsem)
        d1.wait()
        d2.wait()
        d3.wait()

        def _grp(g, fill):
            sl = pl.ds(g * 16, 16)
            r16 = rv_v[sl]
            c16 = cv_v[sl]
            w16 = wv_v[sl]
            rel = c16 - lo
            msk = (rel >= 0) & (rel < NB)
            mi = msk.astype(jnp.int32)
            csum = plsc.cumsum(mi)
            pos = jnp.clip(fill + csum - mi, 0, CAPW - 1)
            nrm = plsc.load_gather(dis_v, [r16]) * w16 \
                * plsc.load_gather(dis_v, [c16])
            plsc.store_scatter(sr_v, [pos], r16, mask=msk)
            plsc.store_scatter(sc_v, [pos], jnp.clip(rel, 0, NB - 1),
                               mask=msk)
            plsc.store_scatter(sn_v, [pos], nrm, mask=msk)
            return fill + _bcast16i(csum, 15)

        return jax.lax.fori_loop(0, ECH // 16, _grp, fill)

    fill = jax.lax.fori_loop(0, NSCH, _chunk,
                             jnp.zeros((16,), jnp.int32))

    cnt_v[pl.ds(0, 16)] = jnp.where(lanes == 0, fill, 0)
    pltpu.sync_copy(cnt_v, cnt_hbm.at[cid].at[sid])
    pltpu.sync_copy(sr_v, rb_hbm.at[cid].at[sid])
    pltpu.sync_copy(sc_v, cb_hbm.at[cid].at[sid])
    pltpu.sync_copy(sn_v, nb_hbm.at[cid].at[sid])

    n0 = wid * ND

    @pl.loop(0, ND // 16)
    def _d2(g):
        d16 = dis_v[pl.ds(n0 + g * 16, 16)]
        d2_v[pl.ds(g * 16, 16)] = d16 * d16

    pltpu.sync_copy(d2_v, dis2_hbm.at[pl.ds(n0, ND)])


@functools.partial(
    pl.kernel,
    out_type=jax.ShapeDtypeStruct((2, NP_, 128), jnp.float32),
    mesh=_mesh,
    compiler_params=_SC_CP,
    scratch_types=[
        pltpu.VMEM((CH,), jnp.int32),
        pltpu.VMEM((CH,), jnp.int32),
        pltpu.VMEM((CH,), jnp.float32),
        pltpu.VMEM((CH, 128), jnp.float32),
        pltpu.VMEM((128, 128), jnp.float32),
        pltpu.VMEM((NB,), jnp.float32),
        pltpu.VMEM((16,), jnp.int32),
        pltpu.VMEM_SHARED((NP_, 128), jnp.float32),
        pltpu.SemaphoreType.DMA,
    ],
)
def _sc_spmm(u_hbm, rb_hbm, cb_hbm, nb_hbm, cnt_hbm, dis2_hbm,
             m_hbm,
             ridx_v, cidx_v, nrm_v, rows_v, ioa_v, d2_v, cnt_v, acc_sh,
             sem):
    cid = jax.lax.axis_index("c")
    sid = jax.lax.axis_index("s")
    row0 = sid * NB


    @pl.loop(0, 128)
    def _z(i):
        @pl.loop(0, 8)
        def _zj(j):
            ioa_v[i, pl.ds(j * 16, 16)] = _z16f()

    @pl.loop(0, 5)
    def _zc(blk):
        pltpu.sync_copy(ioa_v, acc_sh.at[pl.ds(row0 + blk * 128, 128)])

    for half in range(2):
        pltpu.sync_copy(cnt_hbm.at[half].at[sid], cnt_v)
        nch = (jnp.sum(cnt_v[pl.ds(0, 16)]) + (CH - 1)) // CH

        @pl.loop(0, nch)
        def _edge(k):
            e0 = k * CH
            c1 = pltpu.async_copy(
                rb_hbm.at[half].at[sid].at[pl.ds(e0, CH)], ridx_v, sem)
            c2 = pltpu.async_copy(
                cb_hbm.at[half].at[sid].at[pl.ds(e0, CH)], cidx_v, sem)
            c3 = pltpu.async_copy(
                nb_hbm.at[half].at[sid].at[pl.ds(e0, CH)], nrm_v, sem)
            c1.wait()
            c2.wait()
            c3.wait()
            pltpu.async_copy(u_hbm.at[cid].at[ridx_v], rows_v, sem).wait()

            @pl.loop(0, 8)
            def _ofs(g):
                sl = pl.ds(g * 16, 16)
                cidx_v[sl] = cidx_v[sl] + row0

            @pl.loop(0, 8)
            def _g(g):
                n16 = nrm_v[pl.ds(g * 16, 16)]
                for b in range(16):
                    sc = _bcast16(n16, b)
                    row = g * 16 + b
                    for j in range(8):
                        sl = pl.ds(j * 16, 16)
                        rows_v[row, sl] = rows_v[row, sl] * sc

            pltpu.sync_copy(rows_v, acc_sh.at[cidx_v], add=True)

    pltpu.sync_copy(dis2_hbm.at[pl.ds(row0, NB)], d2_v)

    @pl.loop(0, 5)
    def _out(blk):
        r0 = blk * 128
        pltpu.sync_copy(acc_sh.at[pl.ds(row0 + r0, 128)], ioa_v)
        pltpu.sync_copy(u_hbm.at[cid].at[pl.ds(row0 + r0, 128)], rows_v)

        @pl.loop(0, 128)
        def _r(i):
            d2b = plsc.load_gather(
                d2_v, [jnp.zeros((16,), jnp.int32) + (r0 + i)])
            for j in range(8):
                sl = pl.ds(j * 16, 16)
                ioa_v[i, sl] = ioa_v[i, sl] + d2b * rows_v[i, sl]

        pltpu.sync_copy(ioa_v, m_hbm.at[cid].at[pl.ds(row0 + r0, 128)])


def _tc_in_body(x_ref, g_ref, be_ref, w_ref, u_ref):
    h = _bn(x_ref[...], g_ref[...], be_ref[...])
    u = jnp.dot(h, w_ref[...], preferred_element_type=jnp.float32)
    u_ref[0] = jnp.pad(u[:, :128], ((0, NP_ - N), (0, 0)))
    u_ref[1] = jnp.pad(u[:, 128:], ((0, NP_ - N), (0, MIDP - MID)))


def _tc_mid_body(m_ref, b_ref, g_ref, be_ref, w_ref, un_ref):
    m = jnp.concatenate([m_ref[0][:N], m_ref[1][:N, :MID - 128]], axis=1)
    a = _lrelu(m + b_ref[...])
    h = _bn(a, g_ref[...], be_ref[...])
    un = jnp.dot(h, w_ref[...], preferred_element_type=jnp.float32)
    un_ref[0] = jnp.pad(un[:, :128], ((0, NP_ - N), (0, 0)))
    un_ref[1] = jnp.pad(un[:, 128:], ((0, NP_ - N), (0, MIDP - MID)))


def _tc_fin_body(m_ref, b_ref, g_ref, be_ref, batch_ref,
                 wa_ref, ba_ref, g0_ref, be0_ref,
                 wf1_ref, bf1_ref, g4_ref, be4_ref,
                 wf2_ref, bf2_ref, g5_ref, be5_ref,
                 wf3_ref, bf3_ref, z_ref):
    m = jnp.concatenate([m_ref[0][:N], m_ref[1][:N, :MID - 128]], axis=1)
    a = _lrelu(m + b_ref[...])
    h = _bn(a, g_ref[...], be_ref[...])
    s = jnp.dot(h, wa_ref[...], preferred_element_type=jnp.float32) + ba_ref[...]
    oh = (batch_ref[...] == jax.lax.broadcasted_iota(jnp.int32, (1, G), 1)
          ).astype(jnp.float32)
    mg = jnp.max(jnp.where(oh > 0, s, -1e30), axis=0, keepdims=True)
    row_max = _dot3(oh, mg.T)
    e = jnp.exp(s - row_max)
    zsum = _dot3(oh, e, (((0,), (0,)), ((), ())))
    row_z = _dot3(oh, zsum)
    att = e / (row_z + 1e-16)
    pooled = _dot3(oh, h * att, (((0,), (0,)), ((), ())))
    q = _bn(pooled, g0_ref[...], be0_ref[...])
    q = _lrelu(jnp.dot(q, wf1_ref[...], preferred_element_type=jnp.float32)
               + bf1_ref[...])
    q = _bn(q, g4_ref[...], be4_ref[...])
    q = _lrelu(jnp.dot(q, wf2_ref[...], preferred_element_type=jnp.float32)
               + bf2_ref[...])
    q = _bn(q, g5_ref[...], be5_ref[...])
    z_ref[...] = (jnp.dot(q, wf3_ref[...], preferred_element_type=jnp.float32)
                  + bf3_ref[...])


def _call_tc(body, out_shape, *args):
    return pl.pallas_call(
        body,
        out_shape=jax.ShapeDtypeStruct(*out_shape),
    )(*args)


def kernel(x, edge_index, edge_attr, batch, params):
    p = params
    row, col = edge_index[0], edge_index[1]

    r2 = lambda v: v.reshape(1, -1)

    npad = EP - row.shape[0]
    fill = (jnp.arange(npad, dtype=jnp.int32) * 7) % N
    r_pad = jnp.concatenate([row, fill])
    c_pad = jnp.concatenate([col, (fill * 13) % N])
    w_pad = jnp.concatenate([edge_attr, jnp.zeros((npad,), jnp.float32)])

    rb, cb, nb, cnt, dis2 = _sc_pre(r_pad, c_pad, w_pad)

    u = _call_tc(_tc_in_body, ((2, NP_, 128), jnp.float32),
                 x, r2(p['g_in']), r2(p['be_in']), p['W1'])

    for Wn, b, g, be in ((p['W2'], p['b1'], p['g1'], p['be1']),
                         (p['W3'], p['b2'], p['g2'], p['be2'])):
        m = _sc_spmm(u, rb, cb, nb, cnt, dis2)
        u = _call_tc(_tc_mid_body, ((2, NP_, 128), jnp.float32),
                     m, r2(b), r2(g), r2(be), Wn)

    m = _sc_spmm(u, rb, cb, nb, cnt, dis2)
    z = _call_tc(_tc_fin_body, ((G, OUT), jnp.float32),
                 m, r2(p['b3']), r2(p['g3']), r2(p['be3']),
                 batch.reshape(N, 1),
                 p['Wa'], r2(p['ba']), r2(p['g0']), r2(p['be0']),
                 p['Wf1'], r2(p['bf1']), r2(p['g4']), r2(p['be4']),
                 p['Wf2'], r2(p['bf2']), r2(p['g5']), r2(p['be5']),
                 p['Wf3'], r2(p['bf3']))
    return z


# speedup vs baseline: 7.2107x; 1.1660x over previous
import dataclasses
import functools

import jax
import jax.numpy as jnp
from jax.experimental import pallas as pl
from jax.experimental.pallas import tpu as pltpu
from jax.experimental.pallas import tpu_sc as plsc

N = 10000
IN = 128
MID = 240
MIDP = 256
FC = 40
OUT = 2
G = 64

NP_ = 10240
CH = 128
NCH = 158
EP = 16 * NCH * CH
EW = EP // 32
ND = NP_ // 32

_mesh = plsc.VectorSubcoreMesh(core_axis_name="c", subcore_axis_name="s")

_SC_CP = pltpu.CompilerParams()
if "needs_layout_passes" in pltpu.CompilerParams.__dataclass_fields__:
    _SC_CP = dataclasses.replace(_SC_CP, needs_layout_passes=False)


def _dot3(x, w, dn=None):
    xh = x.astype(jnp.bfloat16)
    xl = (x - xh.astype(jnp.float32)).astype(jnp.bfloat16)
    wh = w.astype(jnp.bfloat16)
    wl = (w - wh.astype(jnp.float32)).astype(jnp.bfloat16)
    if dn is None:
        d = lambda a, b: jnp.dot(a, b, preferred_element_type=jnp.float32)
    else:
        d = lambda a, b: jax.lax.dot_general(a, b, dn,
                                             preferred_element_type=jnp.float32)
    return d(xh, wh) + d(xh, wl) + d(xl, wh)


def _bn(h, g, b):
    mu = jnp.mean(h, axis=0, keepdims=True)
    var = jnp.mean((h - mu) ** 2, axis=0, keepdims=True)
    return g * (h - mu) * jax.lax.rsqrt(var + 1e-5) + b


def _lrelu(x):
    return jnp.where(x >= 0, x, 0.01 * x)


def _qrsqrt(x):
    xi = jax.lax.bitcast_convert_type(x, jnp.int32)
    yi = jnp.int32(0x5F3759DF) - (xi >> 1)
    y = jax.lax.bitcast_convert_type(yi, jnp.float32)
    y = y * (1.5 - 0.5 * x * y * y)
    y = y * (1.5 - 0.5 * x * y * y)
    y = y * (1.5 - 0.5 * x * y * y)
    return y


_GDN = jax.lax.GatherDimensionNumbers(
    offset_dims=(), collapsed_slice_dims=(0,), start_index_map=(0,))


def _bcast16(v16, i):
    idx = (jnp.zeros((16,), jnp.int32) + i).reshape(16, 1)
    return jax.lax.gather(v16, idx, _GDN, (1,),
                          mode=jax.lax.GatherScatterMode.PROMISE_IN_BOUNDS)


def _bcast16i(v16, i):
    idx = (jnp.zeros((16,), jnp.int32) + i).reshape(16, 1)
    return jax.lax.gather(v16, idx, _GDN, (1,),
                          mode=jax.lax.GatherScatterMode.PROMISE_IN_BOUNDS)


def _z16f():
    return jnp.zeros((16,), jnp.float32)


NB = NP_ // 16
ECH = 2048
CAPW = 16384
EH = EP // 2
NSCH = EH // ECH


@functools.partial(
    pl.kernel,
    out_type=[jax.ShapeDtypeStruct((2, 16, CAPW), jnp.int32),
              jax.ShapeDtypeStruct((2, 16, CAPW), jnp.int32),
              jax.ShapeDtypeStruct((2, 16, CAPW), jnp.float32),
              jax.ShapeDtypeStruct((2, 16, 16), jnp.int32),
              jax.ShapeDtypeStruct((NP_,), jnp.float32)],
    mesh=_mesh,
    compiler_params=_SC_CP,
    scratch_types=[
        pltpu.VMEM((ECH,), jnp.int32),
        pltpu.VMEM((ECH,), jnp.int32),
        pltpu.VMEM((ECH,), jnp.float32),
        pltpu.VMEM((16, NB), jnp.float32),
        pltpu.VMEM((NB,), jnp.float32),
        pltpu.VMEM((NP_,), jnp.float32),
        pltpu.VMEM((CAPW,), jnp.int32),
        pltpu.VMEM((CAPW,), jnp.int32),
        pltpu.VMEM((CAPW,), jnp.float32),
        pltpu.VMEM((16,), jnp.int32),
        pltpu.VMEM((ND,), jnp.float32),
        pltpu.VMEM_SHARED((NP_,), jnp.float32),
        pltpu.SemaphoreType.DMA,
    ],
)
def _sc_pre(r_hbm, c_hbm, w_hbm,
            rb_hbm, cb_hbm, nb_hbm, cnt_hbm, dis2_hbm,
            rv_v, cv_v, wv_v, acc_v, dt_v, dis_v,
            sr_v, sc_v, sn_v, cnt_v, d2_v, dis_sh, dsem):
    cid = jax.lax.axis_index("c")
    sid = jax.lax.axis_index("s")
    wid = cid * 16 + sid
    lo = sid * NB
    lanes = jnp.arange(16, dtype=jnp.int32)

    @pl.loop(0, 16)
    def _zl(l):
        @pl.loop(0, NB // 16)
        def _zg(g):
            acc_v[l, pl.ds(g * 16, 16)] = _z16f()

    z16i = jnp.zeros((16,), jnp.int32)

    @pl.loop(0, CAPW // 16)
    def _zs(g):
        sl = pl.ds(g * 16, 16)
        sr_v[sl] = z16i
        sc_v[sl] = z16i
        sn_v[sl] = _z16f()

    @pl.loop(0, EP // ECH)
    def _deg(k):
        e0 = k * ECH
        d1 = pltpu.async_copy(c_hbm.at[pl.ds(e0, ECH)], cv_v, dsem)
        d2 = pltpu.async_copy(w_hbm.at[pl.ds(e0, ECH)], wv_v, dsem)
        d1.wait()
        d2.wait()

        @pl.loop(0, ECH // 16, step=4)
        def _g(g0):
            for gg in range(4):
                g = g0 + gg
                c16 = cv_v[pl.ds(g * 16, 16)]
                w16 = wv_v[pl.ds(g * 16, 16)]
                rel = c16 - lo
                msk = (rel >= 0) & (rel < NB)
                idxc = jnp.clip(rel, 0, NB - 1)
                plsc.addupdate_scatter(acc_v, [lanes, idxc], w16, mask=msk)

    @pl.loop(0, NB // 16)
    def _dis(g):
        sl = pl.ds(g * 16, 16)
        tot = acc_v[0, sl]
        for l in range(1, 16):
            tot = tot + acc_v[l, sl]
        dt_v[sl] = _qrsqrt(tot + 1.0)

    pltpu.sync_copy(dt_v, dis_sh.at[pl.ds(lo, NB)])
    plsc.subcore_barrier()
    pltpu.sync_copy(dis_sh, dis_v)

    def _chunk(k, fill):
        e0 = cid * EH + k * ECH
        d1 = pltpu.async_copy(r_hbm.at[pl.ds(e0, ECH)], rv_v, dsem)
        d2 = pltpu.async_copy(c_hbm.at[pl.ds(e0, ECH)], cv_v, dsem)
        d3 = pltpu.async_copy(w_hbm.at[pl.ds(e0, ECH)], wv_v, dsem)
        d1.wait()
        d2.wait()
        d3.wait()

        def _grp(g, fill):
            sl = pl.ds(g * 16, 16)
            r16 = rv_v[sl]
            c16 = cv_v[sl]
            w16 = wv_v[sl]
            rel = c16 - lo
            msk = (rel >= 0) & (rel < NB)
            mi = msk.astype(jnp.int32)
            csum = plsc.cumsum(mi)
            pos = jnp.clip(fill + csum - mi, 0, CAPW - 1)
            nrm = plsc.load_gather(dis_v, [r16]) * w16 \
                * plsc.load_gather(dis_v, [c16])
            plsc.store_scatter(sr_v, [pos], r16, mask=msk)
            plsc.store_scatter(sc_v, [pos], jnp.clip(rel, 0, NB - 1),
                               mask=msk)
            plsc.store_scatter(sn_v, [pos], nrm, mask=msk)
            return fill + _bcast16i(csum, 15)

        return jax.lax.fori_loop(0, ECH // 16, _grp, fill)

    fill = jax.lax.fori_loop(0, NSCH, _chunk,
                             jnp.zeros((16,), jnp.int32))

    cnt_v[pl.ds(0, 16)] = jnp.where(lanes == 0, fill, 0)
    pltpu.sync_copy(cnt_v, cnt_hbm.at[cid].at[sid])
    pltpu.sync_copy(sr_v, rb_hbm.at[cid].at[sid])
    pltpu.sync_copy(sc_v, cb_hbm.at[cid].at[sid])
    pltpu.sync_copy(sn_v, nb_hbm.at[cid].at[sid])

    n0 = wid * ND

    @pl.loop(0, ND // 16)
    def _d2(g):
        d16 = dis_v[pl.ds(n0 + g * 16, 16)]
        d2_v[pl.ds(g * 16, 16)] = d16 * d16

    pltpu.sync_copy(d2_v, dis2_hbm.at[pl.ds(n0, ND)])


@functools.partial(
    pl.kernel,
    out_type=jax.ShapeDtypeStruct((2, NP_, 128), jnp.float32),
    mesh=_mesh,
    compiler_params=_SC_CP,
    scratch_types=[
        pltpu.VMEM((CH,), jnp.int32),
        pltpu.VMEM((CH,), jnp.int32),
        pltpu.VMEM((CH,), jnp.float32),
        pltpu.VMEM((CH,), jnp.int32),
        pltpu.VMEM((CH,), jnp.int32),
        pltpu.VMEM((CH,), jnp.float32),
        pltpu.VMEM((CH, 128), jnp.float32),
        pltpu.VMEM((CH, 128), jnp.float32),
        pltpu.VMEM((NB,), jnp.float32),
        pltpu.VMEM((16,), jnp.int32),
        pltpu.VMEM_SHARED((NP_, 128), jnp.float32),
        pltpu.SemaphoreType.DMA,
        pltpu.SemaphoreType.DMA,
    ],
)
def _sc_spmm(u_hbm, rb_hbm, cb_hbm, nb_hbm, cnt_hbm, dis2_hbm,
             m_hbm,
             ridxa_v, cidxa_v, nrma_v, ridxb_v, cidxb_v, nrmb_v,
             rowsa_v, rowsb_v, d2_v, cnt_v, acc_sh, sema, semb):
    cid = jax.lax.axis_index("c")
    sid = jax.lax.axis_index("s")
    row0 = sid * NB

    @pl.loop(0, 128)
    def _z(i):
        @pl.loop(0, 8)
        def _zj(j):
            rowsa_v[i, pl.ds(j * 16, 16)] = _z16f()

    @pl.loop(0, 5)
    def _zc(blk):
        pltpu.sync_copy(rowsa_v, acc_sh.at[pl.ds(row0 + blk * 128, 128)])

    def _fetch_idx(half, e0, ridx, cidx, nrm, sem):
        c1 = pltpu.async_copy(
            rb_hbm.at[half].at[sid].at[pl.ds(e0, CH)], ridx, sem)
        c2 = pltpu.async_copy(
            cb_hbm.at[half].at[sid].at[pl.ds(e0, CH)], cidx, sem)
        c3 = pltpu.async_copy(
            nb_hbm.at[half].at[sid].at[pl.ds(e0, CH)], nrm, sem)
        c1.wait()
        c2.wait()
        c3.wait()

    def _scale_scatter(ridx, cidx, nrm, rows, sem):
        pltpu.make_async_copy(u_hbm.at[cid].at[ridx], rows, sem).wait()

        @pl.loop(0, 8)
        def _ofs(g):
            sl = pl.ds(g * 16, 16)
            cidx[sl] = cidx[sl] + row0

        @pl.loop(0, 8)
        def _g(g):
            n16 = nrm[pl.ds(g * 16, 16)]
            for b in range(16):
                sc = _bcast16(n16, b)
                row = g * 16 + b
                for j in range(8):
                    sl = pl.ds(j * 16, 16)
                    rows[row, sl] = rows[row, sl] * sc

        pltpu.sync_copy(rows, acc_sh.at[cidx], add=True)

    for half in range(2):
        pltpu.sync_copy(cnt_hbm.at[half].at[sid], cnt_v)
        n = jnp.sum(cnt_v[pl.ds(0, 16)])
        npair = (n + (2 * CH - 1)) // (2 * CH)

        @pl.when(npair > 0)
        def _run():
            _fetch_idx(half, 0, ridxa_v, cidxa_v, nrma_v, sema)
            pltpu.async_copy(u_hbm.at[cid].at[ridxa_v], rowsa_v, sema)

            @pl.loop(0, npair)
            def _pair(kk):
                eb = (2 * kk + 1) * CH
                _fetch_idx(half, eb, ridxb_v, cidxb_v, nrmb_v, semb)
                pltpu.async_copy(u_hbm.at[cid].at[ridxb_v], rowsb_v, semb)
                _scale_scatter(ridxa_v, cidxa_v, nrma_v, rowsa_v, sema)

                @pl.when(kk < npair - 1)
                def _pf():
                    ea = (2 * kk + 2) * CH
                    _fetch_idx(half, ea, ridxa_v, cidxa_v, nrma_v, sema)
                    pltpu.async_copy(u_hbm.at[cid].at[ridxa_v], rowsa_v,
                                     sema)

                _scale_scatter(ridxb_v, cidxb_v, nrmb_v, rowsb_v, semb)

    pltpu.sync_copy(dis2_hbm.at[pl.ds(row0, NB)], d2_v)

    @pl.loop(0, 5)
    def _out(blk):
        r0 = blk * 128
        pltpu.sync_copy(acc_sh.at[pl.ds(row0 + r0, 128)], rowsb_v)
        pltpu.sync_copy(u_hbm.at[cid].at[pl.ds(row0 + r0, 128)], rowsa_v)

        @pl.loop(0, 128)
        def _r(i):
            d2b = plsc.load_gather(
                d2_v, [jnp.zeros((16,), jnp.int32) + (r0 + i)])
            for j in range(8):
                sl = pl.ds(j * 16, 16)
                rowsb_v[i, sl] = rowsb_v[i, sl] + d2b * rowsa_v[i, sl]

        pltpu.sync_copy(rowsb_v, m_hbm.at[cid].at[pl.ds(row0 + r0, 128)])


def _tc_in_body(x_ref, g_ref, be_ref, w_ref, u_ref):
    h = _bn(x_ref[...], g_ref[...], be_ref[...])
    u = jnp.dot(h, w_ref[...], preferred_element_type=jnp.float32)
    u_ref[0] = jnp.pad(u[:, :128], ((0, NP_ - N), (0, 0)))
    u_ref[1] = jnp.pad(u[:, 128:], ((0, NP_ - N), (0, MIDP - MID)))


def _tc_mid_body(m_ref, b_ref, g_ref, be_ref, w_ref, un_ref):
    m = jnp.concatenate([m_ref[0][:N], m_ref[1][:N, :MID - 128]], axis=1)
    a = _lrelu(m + b_ref[...])
    h = _bn(a, g_ref[...], be_ref[...])
    un = jnp.dot(h, w_ref[...], preferred_element_type=jnp.float32)
    un_ref[0] = jnp.pad(un[:, :128], ((0, NP_ - N), (0, 0)))
    un_ref[1] = jnp.pad(un[:, 128:], ((0, NP_ - N), (0, MIDP - MID)))


def _tc_fin_body(m_ref, b_ref, g_ref, be_ref, batch_ref,
                 wa_ref, ba_ref, g0_ref, be0_ref,
                 wf1_ref, bf1_ref, g4_ref, be4_ref,
                 wf2_ref, bf2_ref, g5_ref, be5_ref,
                 wf3_ref, bf3_ref, z_ref):
    m = jnp.concatenate([m_ref[0][:N], m_ref[1][:N, :MID - 128]], axis=1)
    a = _lrelu(m + b_ref[...])
    h = _bn(a, g_ref[...], be_ref[...])
    s = jnp.dot(h, wa_ref[...], preferred_element_type=jnp.float32) + ba_ref[...]
    oh = (batch_ref[...] == jax.lax.broadcasted_iota(jnp.int32, (1, G), 1)
          ).astype(jnp.float32)
    mg = jnp.max(jnp.where(oh > 0, s, -1e30), axis=0, keepdims=True)
    row_max = _dot3(oh, mg.T)
    e = jnp.exp(s - row_max)
    zsum = _dot3(oh, e, (((0,), (0,)), ((), ())))
    row_z = _dot3(oh, zsum)
    att = e / (row_z + 1e-16)
    pooled = _dot3(oh, h * att, (((0,), (0,)), ((), ())))
    q = _bn(pooled, g0_ref[...], be0_ref[...])
    q = _lrelu(jnp.dot(q, wf1_ref[...], preferred_element_type=jnp.float32)
               + bf1_ref[...])
    q = _bn(q, g4_ref[...], be4_ref[...])
    q = _lrelu(jnp.dot(q, wf2_ref[...], preferred_element_type=jnp.float32)
               + bf2_ref[...])
    q = _bn(q, g5_ref[...], be5_ref[...])
    z_ref[...] = (jnp.dot(q, wf3_ref[...], preferred_element_type=jnp.float32)
                  + bf3_ref[...])


def _call_tc(body, out_shape, *args):
    return pl.pallas_call(
        body,
        out_shape=jax.ShapeDtypeStruct(*out_shape),
    )(*args)


def kernel(x, edge_index, edge_attr, batch, params):
    p = params
    row, col = edge_index[0], edge_index[1]

    r2 = lambda v: v.reshape(1, -1)

    npad = EP - row.shape[0]
    fill = (jnp.arange(npad, dtype=jnp.int32) * 7) % N
    r_pad = jnp.concatenate([row, fill])
    c_pad = jnp.concatenate([col, (fill * 13) % N])
    w_pad = jnp.concatenate([edge_attr, jnp.zeros((npad,), jnp.float32)])

    rb, cb, nb, cnt, dis2 = _sc_pre(r_pad, c_pad, w_pad)

    u = _call_tc(_tc_in_body, ((2, NP_, 128), jnp.float32),
                 x, r2(p['g_in']), r2(p['be_in']), p['W1'])

    for Wn, b, g, be in ((p['W2'], p['b1'], p['g1'], p['be1']),
                         (p['W3'], p['b2'], p['g2'], p['be2'])):
        m = _sc_spmm(u, rb, cb, nb, cnt, dis2)
        u = _call_tc(_tc_mid_body, ((2, NP_, 128), jnp.float32),
                     m, r2(b), r2(g), r2(be), Wn)

    m = _sc_spmm(u, rb, cb, nb, cnt, dis2)
    z = _call_tc(_tc_fin_body, ((G, OUT), jnp.float32),
                 m, r2(p['b3']), r2(p['g3']), r2(p['be3']),
                 batch.reshape(N, 1),
                 p['Wa'], r2(p['ba']), r2(p['g0']), r2(p['be0']),
                 p['Wf1'], r2(p['bf1']), r2(p['g4']), r2(p['be4']),
                 p['Wf2'], r2(p['bf2']), r2(p['g5']), r2(p['be5']),
                 p['Wf3'], r2(p['bf3']))
    return z
